# baseline (speedup 1.0000x reference)
import functools

import jax
import jax.numpy as jnp
from jax import lax
from jax.experimental import pallas as pl
from jax.experimental.pallas import tpu as pltpu
from jax.experimental.pallas import tpu_sc as plsc

N_NODES = 10000
N_EDGES = 2048
DIM = 128

_SC_CORES = 2
_SC_SUBCORES = 16
_NW = _SC_CORES * _SC_SUBCORES


def _sc_mesh():
    return plsc.VectorSubcoreMesh(core_axis_name="c", subcore_axis_name="s")


_GB = 2 * N_EDGES
_GPW = _GB // _NW


@functools.cache
def _sc_gather_kernel():
    @functools.partial(
        pl.kernel,
        mesh=_sc_mesh(),
        out_type=jax.ShapeDtypeStruct((_GB, DIM), jnp.float32),
        scratch_types=[
            pltpu.VMEM((_GPW,), jnp.int32),
            pltpu.VMEM((_GPW, DIM), jnp.float32),
            pltpu.SemaphoreType.DMA,
        ],
    )
    def _sc_gather(table_hbm, idx_hbm, out_hbm, idx_v, rows_v, sem):
        wid = lax.axis_index("s") * _SC_CORES + lax.axis_index("c")
        base = wid * _GPW
        pltpu.sync_copy(idx_hbm.at[pl.ds(base, _GPW)], idx_v)
        pltpu.async_copy(table_hbm.at[idx_v], rows_v, sem).wait()
        pltpu.sync_copy(rows_v, out_hbm.at[pl.ds(base, _GPW)])

    return _sc_gather


_HALF = N_NODES // _SC_CORES
_ACC_ROWS = _HALF + 8
_EPT = N_EDGES // _SC_SUBCORES
_AGGC_ROWS = N_EDGES + 8


@functools.cache
def _sc_scatter_kernel():
    @functools.partial(
        pl.kernel,
        mesh=_sc_mesh(),
        out_type=jax.ShapeDtypeStruct((_AGGC_ROWS, DIM), jnp.float32),
        scratch_types=[
            pltpu.VMEM((_EPT,), jnp.int32),
            pltpu.VMEM((_EPT,), jnp.int32),
            pltpu.VMEM((_EPT,), jnp.int32),
            pltpu.VMEM((_EPT, DIM), jnp.float32),
            pltpu.VMEM((_EPT, DIM), jnp.float32),
            pltpu.VMEM((_EPT, DIM), jnp.float32),
            pltpu.VMEM_SHARED((_ACC_ROWS, DIM), jnp.float32),
            pltpu.SemaphoreType.DMA,
            pltpu.SemaphoreType.DMA,
        ],
    )
    def _sc_scatter(msg_hbm, tgt_hbm, out_hbm, idx_v, idx2_v, pos_v, rows_v,
                    rows2_v, zbuf_v, acc_sh, sem_i, sem_m):
        c = lax.axis_index("c")
        s = lax.axis_index("s")
        base = s * _EPT
        cp_i = pltpu.async_copy(tgt_hbm.at[pl.ds(base, _EPT)], idx_v, sem_i)
        cp_m = pltpu.async_copy(msg_hbm.at[pl.ds(base, _EPT)], rows_v, sem_m)

        def _zrow(i, carry):
            for j in range(DIM // 16):
                zbuf_v[i, pl.ds(j * 16, 16)] = jnp.zeros((16,), jnp.float32)
            return carry
        lax.fori_loop(0, _EPT, _zrow, 0)

        cp_i.wait()
        lo = c * _HALF
        for j in range(_EPT // 16):
            v = idx_v[pl.ds(j * 16, 16)] - lo
            inr = (v >= 0) & (v < _HALF)
            idx2_v[pl.ds(j * 16, 16)] = jnp.where(inr, v, _HALF)
            slot = base + j * 16 + lax.iota(jnp.int32, 16)
            pos_v[pl.ds(j * 16, 16)] = jnp.where(inr, slot, N_EDGES)

        pltpu.sync_copy(zbuf_v, acc_sh.at[idx2_v])
        plsc.subcore_barrier()
        cp_m.wait()
        pltpu.sync_copy(rows_v, acc_sh.at[idx2_v], add=True)
        plsc.subcore_barrier()

        pltpu.sync_copy(acc_sh.at[idx2_v], rows2_v)
        pltpu.sync_copy(rows2_v, out_hbm.at[pos_v])

    return _sc_scatter


_RPW = N_EDGES // _NW


@functools.cache
def _sc_rowwrite_kernel():
    @functools.partial(
        pl.kernel,
        mesh=_sc_mesh(),
        out_type=(),
        scratch_types=[
            pltpu.VMEM((_RPW,), jnp.int32),
            pltpu.VMEM((_RPW, DIM), jnp.float32),
        ],
    )
    def _sc_rowwrite(x_hbm, xfix_hbm, tgt_hbm, idx_v, rows_v):
        wid = lax.axis_index("s") * _SC_CORES + lax.axis_index("c")
        base = wid * _RPW
        pltpu.sync_copy(tgt_hbm.at[pl.ds(base, _RPW)], idx_v)
        pltpu.sync_copy(xfix_hbm.at[pl.ds(base, _RPW)], rows_v)
        pltpu.sync_copy(rows_v, x_hbm.at[idx_v])

    return _sc_rowwrite


def _apply_rowwrite(xbg, xfix, tgt):
    xr = jax.new_ref(xbg)
    _sc_rowwrite_kernel()(xr, xfix, tgt)
    return xr[...]


def _edge_mlp_body(rows_ref, rel_ref, w1a, w1b, b1, w2, b2, out_ref):
    xs = rows_ref[:N_EDGES, :].astype(jnp.bfloat16)
    xt = rows_ref[N_EDGES:, :].astype(jnp.bfloat16)
    h = jnp.dot(xs, w1a[...].astype(jnp.bfloat16),
                preferred_element_type=jnp.float32)
    h = h + jnp.dot(xt, w1b[...].astype(jnp.bfloat16),
                    preferred_element_type=jnp.float32)
    h = jnp.maximum(h + b1[...], 0.0).astype(jnp.bfloat16)
    msg = jnp.dot(h, w2[...].astype(jnp.bfloat16),
                  preferred_element_type=jnp.float32) + b2[...]
    src = rel_ref[:, 0:1]
    tgt = rel_ref[:, 2:3]
    valid = ((src < N_NODES) & (tgt < N_NODES)).astype(jnp.float32)
    out_ref[...] = msg * valid


def _edge_mlp(rows, relations, W1, b1, W2, b2):
    return pl.pallas_call(
        _edge_mlp_body,
        out_shape=jax.ShapeDtypeStruct((N_EDGES, DIM), jnp.float32),
    )(rows, relations, W1[:DIM], W1[DIM:], b1.reshape(1, DIM), W2,
      b2.reshape(1, DIM))


def _bg_gru_body(x_ref, bih, whh, bhh, out_ref):
    gh = jnp.dot(x_ref[...].astype(jnp.bfloat16),
                 whh[...].astype(jnp.bfloat16),
                 preferred_element_type=jnp.float32) + bhh[...]
    gi = bih[...]
    r = jax.nn.sigmoid(gi[:, :DIM] + gh[:, :DIM])
    z = jax.nn.sigmoid(gi[:, DIM:2 * DIM] + gh[:, DIM:2 * DIM])
    n = jnp.tanh(gi[:, 2 * DIM:] + r * gh[:, 2 * DIM:])
    out_ref[...] = (1.0 - z) * n + z * x_ref[...]


def _bg_gru(x, bih, Whh, bhh):
    R = 2000
    full = lambda i: (0, 0)
    return pl.pallas_call(
        _bg_gru_body,
        grid=(N_NODES // R,),
        in_specs=[
            pl.BlockSpec((R, DIM), lambda i: (i, 0)),
            pl.BlockSpec((1, 3 * DIM), full),
            pl.BlockSpec((DIM, 3 * DIM), full),
            pl.BlockSpec((1, 3 * DIM), full),
        ],
        out_specs=pl.BlockSpec((R, DIM), lambda i: (i, 0)),
        out_shape=jax.ShapeDtypeStruct((N_NODES, DIM), jnp.float32),
    )(x, bih.reshape(1, -1), Whh, bhh.reshape(1, -1))


def _fixup_body(xt_ref, aggc_ref, wih, bih, whh, bhh, out_ref):
    agg = aggc_ref[:N_EDGES, :]
    gi = jnp.dot(agg.astype(jnp.bfloat16), wih[...].astype(jnp.bfloat16),
                 preferred_element_type=jnp.float32) + bih[...]
    gh = jnp.dot(xt_ref[...].astype(jnp.bfloat16),
                 whh[...].astype(jnp.bfloat16),
                 preferred_element_type=jnp.float32) + bhh[...]
    r = jax.nn.sigmoid(gi[:, :DIM] + gh[:, :DIM])
    z = jax.nn.sigmoid(gi[:, DIM:2 * DIM] + gh[:, DIM:2 * DIM])
    n = jnp.tanh(gi[:, 2 * DIM:] + r * gh[:, 2 * DIM:])
    out_ref[...] = (1.0 - z) * n + z * xt_ref[...]


def _fixup(rows, aggc, Wih, bih, Whh, bhh):
    full = lambda i: (0, 0)
    return pl.pallas_call(
        _fixup_body,
        grid=(1,),
        in_specs=[
            pl.BlockSpec((N_EDGES, DIM), lambda i: (1, 0)),
            pl.BlockSpec((_AGGC_ROWS, DIM), full),
            pl.BlockSpec((DIM, 3 * DIM), full),
            pl.BlockSpec((1, 3 * DIM), full),
            pl.BlockSpec((DIM, 3 * DIM), full),
            pl.BlockSpec((1, 3 * DIM), full),
        ],
        out_specs=pl.BlockSpec((N_EDGES, DIM), full),
        out_shape=jax.ShapeDtypeStruct((N_EDGES, DIM), jnp.float32),
    )(rows, aggc, Wih, bih.reshape(1, -1), Whh, bhh.reshape(1, -1))


def _layer_norm(h, g, b):
    mu = jnp.mean(h, axis=-1, keepdims=True)
    var = jnp.mean((h - mu) ** 2, axis=-1, keepdims=True)
    return (h - mu) * jax.lax.rsqrt(var + 1e-5) * g + b


def _readout_body(sym_ref, x_ref, d1, db1, g1, c1, d2, db2, g2, c2, d3, db3,
                  out_ref):
    agg = jnp.dot(sym_ref[...].astype(jnp.bfloat16),
                  x_ref[...].astype(jnp.bfloat16),
                  preferred_element_type=jnp.float32)
    h = jnp.dot(agg.astype(jnp.bfloat16), d1[...].astype(jnp.bfloat16),
                preferred_element_type=jnp.float32) + db1[...]
    h = jnp.maximum(_layer_norm(h, g1[...], c1[...]), 0.0)
    h = jnp.dot(h.astype(jnp.bfloat16), d2[...].astype(jnp.bfloat16),
                preferred_element_type=jnp.float32) + db2[...]
    h = jnp.maximum(_layer_norm(h, g2[...], c2[...]), 0.0)
    out_ref[...] = jnp.dot(h.astype(jnp.bfloat16),
                           d3[...].astype(jnp.bfloat16),
                           preferred_element_type=jnp.float32) + db3[...]


def _readout(symbols, x, p):
    B = symbols.shape[0]
    return pl.pallas_call(
        _readout_body,
        out_shape=jax.ShapeDtypeStruct((B, DIM), jnp.float32),
    )(symbols, x,
      p["D1"], p["db1"].reshape(1, -1), p["ln1_g"].reshape(1, -1),
      p["ln1_b"].reshape(1, -1),
      p["D2"], p["db2"].reshape(1, -1), p["ln2_g"].reshape(1, -1),
      p["ln2_b"].reshape(1, -1),
      p["D3"], p["db3"].reshape(1, -1))


def kernel(symbols, relations, params):
    p = params
    x = p["emb"]
    src = relations[:, 0]
    tgt = relations[:, 2]
    idx = jnp.concatenate([src, tgt], axis=0)
    for i in range(3):
        rows = _sc_gather_kernel()(x, idx)
        xbg = _bg_gru(x, p[f"g{i}_bih"], p[f"g{i}_Whh"], p[f"g{i}_bhh"])
        msg = _edge_mlp(rows, relations, p[f"g{i}_W1"], p[f"g{i}_b1"],
                        p[f"g{i}_W2"], p[f"g{i}_b2"])
        aggc = _sc_scatter_kernel()(msg, tgt)
        xfix = _fixup(rows, aggc, p[f"g{i}_Wih"], p[f"g{i}_bih"],
                      p[f"g{i}_Whh"], p[f"g{i}_bhh"])
        x = _apply_rowwrite(xbg, xfix, tgt)
    return _readout(symbols, x, p)


# speedup vs baseline: 2.5223x; 2.5223x over previous
import functools

import jax
import jax.numpy as jnp
from jax import lax
from jax.experimental import pallas as pl
from jax.experimental.pallas import tpu as pltpu
from jax.experimental.pallas import tpu_sc as plsc

N_NODES = 10000
N_EDGES = 2048
DIM = 128

_SC_CORES = 2
_SC_SUBCORES = 16
_NW = _SC_CORES * _SC_SUBCORES


def _sc_mesh():
    return plsc.VectorSubcoreMesh(core_axis_name="c", subcore_axis_name="s")


_GB = 2 * N_EDGES


@functools.cache
def _sc_gather_kernel(n_rows):
    rpw = n_rows // _NW

    @functools.partial(
        pl.kernel,
        mesh=_sc_mesh(),
        out_type=jax.ShapeDtypeStruct((n_rows, DIM), jnp.float32),
        scratch_types=[
            pltpu.VMEM((rpw,), jnp.int32),
            pltpu.VMEM((rpw, DIM), jnp.float32),
            pltpu.SemaphoreType.DMA,
        ],
    )
    def _sc_gather(table_hbm, idx_hbm, out_hbm, idx_v, rows_v, sem):
        wid = lax.axis_index("s") * _SC_CORES + lax.axis_index("c")
        base = wid * rpw
        pltpu.sync_copy(idx_hbm.at[pl.ds(base, rpw)], idx_v)
        pltpu.async_copy(table_hbm.at[idx_v], rows_v, sem).wait()
        pltpu.sync_copy(rows_v, out_hbm.at[pl.ds(base, rpw)])

    return _sc_gather


_HALF = N_NODES // _SC_CORES
_ACC_ROWS = _HALF + 8
_EPT = N_EDGES // _SC_SUBCORES
_CPT = _HALF // _SC_SUBCORES


@functools.cache
def _sc_scatter_kernel():
    @functools.partial(
        pl.kernel,
        mesh=_sc_mesh(),
        out_type=jax.ShapeDtypeStruct((N_NODES, DIM), jnp.float32),
        scratch_types=[
            pltpu.VMEM((_EPT,), jnp.int32),
            pltpu.VMEM((_EPT,), jnp.int32),
            pltpu.VMEM((_EPT, DIM), jnp.float32),
            pltpu.VMEM((_EPT, DIM), jnp.float32),
            pltpu.VMEM_SHARED((_ACC_ROWS, DIM), jnp.float32),
            pltpu.SemaphoreType.DMA,
            pltpu.SemaphoreType.DMA,
        ],
    )
    def _sc_scatter(msg_hbm, tgt_hbm, out_hbm, idx_v, idx2_v, rows_v,
                    zbuf_v, acc_sh, sem_i, sem_m):
        c = lax.axis_index("c")
        s = lax.axis_index("s")
        base = s * _EPT
        cp_i = pltpu.async_copy(tgt_hbm.at[pl.ds(base, _EPT)], idx_v, sem_i)
        cp_m = pltpu.async_copy(msg_hbm.at[pl.ds(base, _EPT)], rows_v, sem_m)

        def _zrow(i, carry):
            for j in range(DIM // 16):
                zbuf_v[i, pl.ds(j * 16, 16)] = jnp.zeros((16,), jnp.float32)
            return carry
        lax.fori_loop(0, _EPT, _zrow, 0)

        cp_i.wait()
        lo = c * _HALF
        for j in range(_EPT // 16):
            v = idx_v[pl.ds(j * 16, 16)] - lo
            inr = (v >= 0) & (v < _HALF)
            idx2_v[pl.ds(j * 16, 16)] = jnp.where(inr, v, _HALF)

        pltpu.sync_copy(zbuf_v, acc_sh.at[idx2_v])
        plsc.subcore_barrier()
        cp_m.wait()
        pltpu.sync_copy(rows_v, acc_sh.at[idx2_v], add=True)
        plsc.subcore_barrier()

        pltpu.sync_copy(acc_sh.at[pl.ds(s * _CPT, _CPT)],
                        out_hbm.at[pl.ds(lo + s * _CPT, _CPT)])

        @pl.when(s == _SC_SUBCORES - 1)
        def _():
            rem = _HALF - _SC_SUBCORES * _CPT
            pltpu.sync_copy(acc_sh.at[pl.ds(_SC_SUBCORES * _CPT, rem)],
                            out_hbm.at[pl.ds(lo + _SC_SUBCORES * _CPT, rem)])

    return _sc_scatter


_RPW = N_EDGES // _NW


@functools.cache
def _sc_rowwrite_kernel():
    @functools.partial(
        pl.kernel,
        mesh=_sc_mesh(),
        out_type=(),
        scratch_types=[
            pltpu.VMEM((_RPW,), jnp.int32),
            pltpu.VMEM((_RPW, DIM), jnp.float32),
        ],
    )
    def _sc_rowwrite(x_hbm, xfix_hbm, tgt_hbm, idx_v, rows_v):
        wid = lax.axis_index("s") * _SC_CORES + lax.axis_index("c")
        base = wid * _RPW
        pltpu.sync_copy(tgt_hbm.at[pl.ds(base, _RPW)], idx_v)
        pltpu.sync_copy(xfix_hbm.at[pl.ds(base, _RPW)], rows_v)
        pltpu.sync_copy(rows_v, x_hbm.at[idx_v])

    return _sc_rowwrite


def _apply_rowwrite(xbg, xfix, tgt):
    xr = jax.new_ref(xbg)
    _sc_rowwrite_kernel()(xr, xfix, tgt)
    return xr[...]


def _edge_mlp_body(rows_ref, rel_ref, w1a, w1b, b1, w2, b2, out_ref):
    xs = rows_ref[:N_EDGES, :].astype(jnp.bfloat16)
    xt = rows_ref[N_EDGES:, :].astype(jnp.bfloat16)
    h = jnp.dot(xs, w1a[...].astype(jnp.bfloat16),
                preferred_element_type=jnp.float32)
    h = h + jnp.dot(xt, w1b[...].astype(jnp.bfloat16),
                    preferred_element_type=jnp.float32)
    h = jnp.maximum(h + b1[...], 0.0).astype(jnp.bfloat16)
    msg = jnp.dot(h, w2[...].astype(jnp.bfloat16),
                  preferred_element_type=jnp.float32) + b2[...]
    src = rel_ref[:, 0:1]
    tgt = rel_ref[:, 2:3]
    valid = ((src < N_NODES) & (tgt < N_NODES)).astype(jnp.float32)
    out_ref[...] = msg * valid


def _edge_mlp(rows, relations, W1, b1, W2, b2):
    return pl.pallas_call(
        _edge_mlp_body,
        out_shape=jax.ShapeDtypeStruct((N_EDGES, DIM), jnp.float32),
    )(rows, relations, W1[:DIM], W1[DIM:], b1.reshape(1, DIM), W2,
      b2.reshape(1, DIM))


def _bg_gru_body(x_ref, bih, whh, bhh, out_ref):
    gh = jnp.dot(x_ref[...].astype(jnp.bfloat16),
                 whh[...].astype(jnp.bfloat16),
                 preferred_element_type=jnp.float32) + bhh[...]
    gi = bih[...]
    r = jax.nn.sigmoid(gi[:, :DIM] + gh[:, :DIM])
    z = jax.nn.sigmoid(gi[:, DIM:2 * DIM] + gh[:, DIM:2 * DIM])
    n = jnp.tanh(gi[:, 2 * DIM:] + r * gh[:, 2 * DIM:])
    out_ref[...] = (1.0 - z) * n + z * x_ref[...]


def _bg_gru(x, bih, Whh, bhh):
    R = 2000
    full = lambda i: (0, 0)
    return pl.pallas_call(
        _bg_gru_body,
        grid=(N_NODES // R,),
        in_specs=[
            pl.BlockSpec((R, DIM), lambda i: (i, 0)),
            pl.BlockSpec((1, 3 * DIM), full),
            pl.BlockSpec((DIM, 3 * DIM), full),
            pl.BlockSpec((1, 3 * DIM), full),
        ],
        out_specs=pl.BlockSpec((R, DIM), lambda i: (i, 0)),
        out_shape=jax.ShapeDtypeStruct((N_NODES, DIM), jnp.float32),
    )(x, bih.reshape(1, -1), Whh, bhh.reshape(1, -1))


def _fixup_body(xt_ref, aggc_ref, wih, bih, whh, bhh, out_ref):
    agg = aggc_ref[...]
    gi = jnp.dot(agg.astype(jnp.bfloat16), wih[...].astype(jnp.bfloat16),
                 preferred_element_type=jnp.float32) + bih[...]
    gh = jnp.dot(xt_ref[...].astype(jnp.bfloat16),
                 whh[...].astype(jnp.bfloat16),
                 preferred_element_type=jnp.float32) + bhh[...]
    r = jax.nn.sigmoid(gi[:, :DIM] + gh[:, :DIM])
    z = jax.nn.sigmoid(gi[:, DIM:2 * DIM] + gh[:, DIM:2 * DIM])
    n = jnp.tanh(gi[:, 2 * DIM:] + r * gh[:, 2 * DIM:])
    out_ref[...] = (1.0 - z) * n + z * xt_ref[...]


def _fixup(rows, aggc, Wih, bih, Whh, bhh):
    full = lambda i: (0, 0)
    return pl.pallas_call(
        _fixup_body,
        grid=(1,),
        in_specs=[
            pl.BlockSpec((N_EDGES, DIM), lambda i: (1, 0)),
            pl.BlockSpec((N_EDGES, DIM), full),
            pl.BlockSpec((DIM, 3 * DIM), full),
            pl.BlockSpec((1, 3 * DIM), full),
            pl.BlockSpec((DIM, 3 * DIM), full),
            pl.BlockSpec((1, 3 * DIM), full),
        ],
        out_specs=pl.BlockSpec((N_EDGES, DIM), full),
        out_shape=jax.ShapeDtypeStruct((N_EDGES, DIM), jnp.float32),
    )(rows, aggc, Wih, bih.reshape(1, -1), Whh, bhh.reshape(1, -1))


def _layer_norm(h, g, b):
    mu = jnp.mean(h, axis=-1, keepdims=True)
    var = jnp.mean((h - mu) ** 2, axis=-1, keepdims=True)
    return (h - mu) * jax.lax.rsqrt(var + 1e-5) * g + b


def _readout_body(sym_ref, x_ref, d1, db1, g1, c1, d2, db2, g2, c2, d3, db3,
                  out_ref):
    agg = jnp.dot(sym_ref[...].astype(jnp.bfloat16),
                  x_ref[...].astype(jnp.bfloat16),
                  preferred_element_type=jnp.float32)
    h = jnp.dot(agg.astype(jnp.bfloat16), d1[...].astype(jnp.bfloat16),
                preferred_element_type=jnp.float32) + db1[...]
    h = jnp.maximum(_layer_norm(h, g1[...], c1[...]), 0.0)
    h = jnp.dot(h.astype(jnp.bfloat16), d2[...].astype(jnp.bfloat16),
                preferred_element_type=jnp.float32) + db2[...]
    h = jnp.maximum(_layer_norm(h, g2[...], c2[...]), 0.0)
    out_ref[...] = jnp.dot(h.astype(jnp.bfloat16),
                           d3[...].astype(jnp.bfloat16),
                           preferred_element_type=jnp.float32) + db3[...]


def _readout(symbols, x, p):
    B = symbols.shape[0]
    return pl.pallas_call(
        _readout_body,
        out_shape=jax.ShapeDtypeStruct((B, DIM), jnp.float32),
    )(symbols, x,
      p["D1"], p["db1"].reshape(1, -1), p["ln1_g"].reshape(1, -1),
      p["ln1_b"].reshape(1, -1),
      p["D2"], p["db2"].reshape(1, -1), p["ln2_g"].reshape(1, -1),
      p["ln2_b"].reshape(1, -1),
      p["D3"], p["db3"].reshape(1, -1))


def kernel(symbols, relations, params):
    p = params
    x = p["emb"]
    src = relations[:, 0]
    tgt = relations[:, 2]
    idx = jnp.concatenate([src, tgt], axis=0)
    for i in range(3):
        rows = _sc_gather_kernel(_GB)(x, idx)
        xbg = _bg_gru(x, p[f"g{i}_bih"], p[f"g{i}_Whh"], p[f"g{i}_bhh"])
        msg = _edge_mlp(rows, relations, p[f"g{i}_W1"], p[f"g{i}_b1"],
                        p[f"g{i}_W2"], p[f"g{i}_b2"])
        agg = _sc_scatter_kernel()(msg, tgt)
        aggc = _sc_gather_kernel(N_EDGES)(agg, tgt)
        xfix = _fixup(rows, aggc, p[f"g{i}_Wih"], p[f"g{i}_bih"],
                      p[f"g{i}_Whh"], p[f"g{i}_bhh"])
        x = _apply_rowwrite(xbg, xfix, tgt)
    return _readout(symbols, x, p)


# speedup vs baseline: 3.1574x; 1.2518x over previous
import functools

import jax
import jax.numpy as jnp
from jax import lax
from jax.experimental import pallas as pl
from jax.experimental.pallas import tpu as pltpu
from jax.experimental.pallas import tpu_sc as plsc

N_NODES = 10000
N_EDGES = 2048
DIM = 128

_SC_CORES = 2
_SC_SUBCORES = 16
_NW = _SC_CORES * _SC_SUBCORES

_GB = 2 * N_EDGES
_GPW = _GB // _NW


@functools.cache
def _sc_gather_kernel(dt):
    dt = jnp.dtype(dt)
    mesh = plsc.VectorSubcoreMesh(core_axis_name="c", subcore_axis_name="s")

    @functools.partial(
        pl.kernel,
        mesh=mesh,
        out_type=jax.ShapeDtypeStruct((_GB, DIM), dt),
        scratch_types=[
            pltpu.VMEM((_GPW,), jnp.int32),
            pltpu.VMEM((_GPW, DIM), dt),
            pltpu.SemaphoreType.DMA,
        ],
    )
    def _sc_gather(table_hbm, src_hbm, tgt_hbm, out_hbm, idx_v, rows_v, sem):
        wid = lax.axis_index("s") * _SC_CORES + lax.axis_index("c")
        base = wid * _GPW
        half = _GB // 2

        @pl.when(base < half)
        def _():
            pltpu.sync_copy(src_hbm.at[pl.ds(base, _GPW)], idx_v)

        @pl.when(base >= half)
        def _():
            pltpu.sync_copy(tgt_hbm.at[pl.ds(base - half, _GPW)], idx_v)

        pltpu.async_copy(table_hbm.at[idx_v], rows_v, sem).wait()
        pltpu.sync_copy(rows_v, out_hbm.at[pl.ds(base, _GPW)])

    return _sc_gather


_HALF = N_NODES // _SC_CORES
_ACC_ROWS = _HALF + 8
_EPT = N_EDGES // _SC_SUBCORES
_ZPT = _ACC_ROWS // _SC_SUBCORES
_CPT = _HALF // _SC_SUBCORES


_ZCH = 64


@functools.cache
def _sc_scatter_kernel():
    mesh = plsc.VectorSubcoreMesh(core_axis_name="c", subcore_axis_name="s")

    @functools.partial(
        pl.kernel,
        mesh=mesh,
        out_type=jax.ShapeDtypeStruct((N_NODES, DIM), jnp.float32),
        scratch_types=[
            pltpu.VMEM((_EPT,), jnp.int32),
            pltpu.VMEM((_EPT,), jnp.int32),
            pltpu.VMEM((_EPT, DIM), jnp.float32),
            pltpu.VMEM((_ZCH, DIM), jnp.float32),
            pltpu.VMEM_SHARED((_ACC_ROWS, DIM), jnp.float32),
            pltpu.SemaphoreType.DMA,
            pltpu.SemaphoreType.DMA,
            pltpu.SemaphoreType.DMA,
        ],
    )
    def _sc_scatter(msg_hbm, tgt_hbm, out_hbm, idx_v, idx2_v, rows_v, zbuf_v,
                    acc_sh, sem_i, sem_m, sem_z):
        c = lax.axis_index("c")
        s = lax.axis_index("s")

        base = s * _EPT
        cp_i = pltpu.async_copy(tgt_hbm.at[pl.ds(base, _EPT)], idx_v, sem_i)
        cp_m = pltpu.async_copy(msg_hbm.at[pl.ds(base, _EPT)], rows_v, sem_m)

        def _zrow(i, carry):
            for j in range(DIM // 16):
                zbuf_v[i, pl.ds(j * 16, 16)] = jnp.zeros((16,), jnp.float32)
            return carry
        lax.fori_loop(0, _ZCH, _zrow, 0)
        zc = []
        for kk in range(_ZPT // _ZCH):
            zc.append(pltpu.async_copy(
                zbuf_v, acc_sh.at[pl.ds(s * _ZPT + kk * _ZCH, _ZCH)], sem_z))
        rem = _ZPT % _ZCH
        zc.append(pltpu.async_copy(
            zbuf_v.at[pl.ds(0, rem)],
            acc_sh.at[pl.ds(s * _ZPT + _ZPT - rem, rem)], sem_z))

        cp_i.wait()
        lo = c * _HALF
        for j in range(_EPT // 16):
            v = idx_v[pl.ds(j * 16, 16)] - lo
            inr = (v >= 0) & (v < _HALF)
            idx2_v[pl.ds(j * 16, 16)] = jnp.where(inr, v, _HALF)

        for z in zc:
            z.wait()
        cp_m.wait()
        plsc.subcore_barrier()
        pltpu.sync_copy(rows_v, acc_sh.at[idx2_v], add=True)
        plsc.subcore_barrier()

        pltpu.sync_copy(acc_sh.at[pl.ds(s * _CPT, _CPT)],
                        out_hbm.at[pl.ds(lo + s * _CPT, _CPT)])

        @pl.when(s == _SC_SUBCORES - 1)
        def _():
            rem = _HALF - _SC_SUBCORES * _CPT
            pltpu.sync_copy(acc_sh.at[pl.ds(_SC_SUBCORES * _CPT, rem)],
                            out_hbm.at[pl.ds(lo + _SC_SUBCORES * _CPT, rem)])

    return _sc_scatter


def _edge_mlp_body(rows_ref, rel_ref, w1a, w1b, b1, w2, b2, out_ref):
    xs = rows_ref[:N_EDGES, :].astype(jnp.bfloat16)
    xt = rows_ref[N_EDGES:, :].astype(jnp.bfloat16)
    h = jnp.dot(xs, w1a[...].astype(jnp.bfloat16),
                preferred_element_type=jnp.float32)
    h = h + jnp.dot(xt, w1b[...].astype(jnp.bfloat16),
                    preferred_element_type=jnp.float32)
    h = jnp.maximum(h + b1[...], 0.0).astype(jnp.bfloat16)
    msg = jnp.dot(h, w2[...].astype(jnp.bfloat16),
                  preferred_element_type=jnp.float32) + b2[...]
    src = rel_ref[:, 0:1]
    tgt = rel_ref[:, 2:3]
    valid = ((src < N_NODES) & (tgt < N_NODES)).astype(jnp.float32)
    out_ref[...] = msg * valid


def _edge_mlp(rows, relations, W1, b1, W2, b2):
    return pl.pallas_call(
        _edge_mlp_body,
        out_shape=jax.ShapeDtypeStruct((N_EDGES, DIM), jnp.float32),
    )(rows, relations, W1[:DIM], W1[DIM:], b1.reshape(1, DIM), W2,
      b2.reshape(1, DIM))


def _gru_body(agg_ref, x_ref, wih, bih, whh, bhh, out_ref):
    gi = jnp.dot(agg_ref[...].astype(jnp.bfloat16),
                 wih[...].astype(jnp.bfloat16),
                 preferred_element_type=jnp.float32) + bih[...]
    gh = jnp.dot(x_ref[...].astype(jnp.bfloat16),
                 whh[...].astype(jnp.bfloat16),
                 preferred_element_type=jnp.float32) + bhh[...]
    r = jax.nn.sigmoid(gi[:, :DIM] + gh[:, :DIM])
    z = jax.nn.sigmoid(gi[:, DIM:2 * DIM] + gh[:, DIM:2 * DIM])
    n = jnp.tanh(gi[:, 2 * DIM:] + r * gh[:, 2 * DIM:])
    x32 = x_ref[...].astype(jnp.float32)
    out_ref[...] = ((1.0 - z) * n + z * x32).astype(out_ref.dtype)


def _gru(agg, x, Wih, bih, Whh, bhh, out_dtype=jnp.float32):
    R = 2000
    full = lambda i: (0, 0)
    return pl.pallas_call(
        _gru_body,
        grid=(N_NODES // R,),
        in_specs=[
            pl.BlockSpec((R, DIM), lambda i: (i, 0)),
            pl.BlockSpec((R, DIM), lambda i: (i, 0)),
            pl.BlockSpec((DIM, 3 * DIM), full),
            pl.BlockSpec((1, 3 * DIM), full),
            pl.BlockSpec((DIM, 3 * DIM), full),
            pl.BlockSpec((1, 3 * DIM), full),
        ],
        out_specs=pl.BlockSpec((R, DIM), lambda i: (i, 0)),
        out_shape=jax.ShapeDtypeStruct((N_NODES, DIM), out_dtype),
    )(agg, x, Wih, bih.reshape(1, -1), Whh, bhh.reshape(1, -1))


def _layer_norm(h, g, b):
    mu = jnp.mean(h, axis=-1, keepdims=True)
    var = jnp.mean((h - mu) ** 2, axis=-1, keepdims=True)
    return (h - mu) * jax.lax.rsqrt(var + 1e-5) * g + b


def _readout_body(sym_ref, x_ref, d1, db1, g1, c1, d2, db2, g2, c2, d3, db3,
                  out_ref):
    agg = jnp.dot(sym_ref[...].astype(jnp.bfloat16),
                  x_ref[...].astype(jnp.bfloat16),
                  preferred_element_type=jnp.float32)
    h = jnp.dot(agg.astype(jnp.bfloat16), d1[...].astype(jnp.bfloat16),
                preferred_element_type=jnp.float32) + db1[...]
    h = jnp.maximum(_layer_norm(h, g1[...], c1[...]), 0.0)
    h = jnp.dot(h.astype(jnp.bfloat16), d2[...].astype(jnp.bfloat16),
                preferred_element_type=jnp.float32) + db2[...]
    h = jnp.maximum(_layer_norm(h, g2[...], c2[...]), 0.0)
    out_ref[...] = jnp.dot(h.astype(jnp.bfloat16),
                           d3[...].astype(jnp.bfloat16),
                           preferred_element_type=jnp.float32) + db3[...]


def _readout(symbols, x, p):
    B = symbols.shape[0]
    return pl.pallas_call(
        _readout_body,
        out_shape=jax.ShapeDtypeStruct((B, DIM), jnp.float32),
    )(symbols, x,
      p["D1"], p["db1"].reshape(1, -1), p["ln1_g"].reshape(1, -1),
      p["ln1_b"].reshape(1, -1),
      p["D2"], p["db2"].reshape(1, -1), p["ln2_g"].reshape(1, -1),
      p["ln2_b"].reshape(1, -1),
      p["D3"], p["db3"].reshape(1, -1))


def kernel(symbols, relations, params):
    p = params
    x = p["emb"]
    src = relations[:, 0]
    tgt = relations[:, 2]
    for i in range(3):
        rows = _sc_gather_kernel(jnp.dtype(x.dtype).name)(x, src, tgt)
        msg = _edge_mlp(rows, relations, p[f"g{i}_W1"], p[f"g{i}_b1"],
                        p[f"g{i}_W2"], p[f"g{i}_b2"])
        agg = _sc_scatter_kernel()(msg, tgt)
        odt = jnp.bfloat16 if i == 2 else jnp.float32
        x = _gru(agg, x, p[f"g{i}_Wih"], p[f"g{i}_bih"], p[f"g{i}_Whh"],
                 p[f"g{i}_bhh"], out_dtype=odt)
    return _readout(symbols, x, p)


# speedup vs baseline: 3.1664x; 1.0028x over previous
import functools

import jax
import jax.numpy as jnp
from jax import lax
from jax.experimental import pallas as pl
from jax.experimental.pallas import tpu as pltpu
from jax.experimental.pallas import tpu_sc as plsc

N_NODES = 10000
N_EDGES = 2048
DIM = 128

_SC_CORES = 2
_SC_SUBCORES = 16
_NW = _SC_CORES * _SC_SUBCORES

_GB = 2 * N_EDGES
_GPW = _GB // _NW


@functools.cache
def _sc_gather_kernel(dt):
    dt = jnp.dtype(dt)
    mesh = plsc.VectorSubcoreMesh(core_axis_name="c", subcore_axis_name="s")

    @functools.partial(
        pl.kernel,
        mesh=mesh,
        out_type=jax.ShapeDtypeStruct((_GB, DIM), dt),
        scratch_types=[
            pltpu.VMEM((_GPW,), jnp.int32),
            pltpu.VMEM((_GPW, DIM), dt),
            pltpu.SemaphoreType.DMA,
        ],
    )
    def _sc_gather(table_hbm, src_hbm, tgt_hbm, out_hbm, idx_v, rows_v, sem):
        wid = lax.axis_index("s") * _SC_CORES + lax.axis_index("c")
        base = wid * _GPW
        half = _GB // 2

        @pl.when(base < half)
        def _():
            pltpu.sync_copy(src_hbm.at[pl.ds(base, _GPW)], idx_v)

        @pl.when(base >= half)
        def _():
            pltpu.sync_copy(tgt_hbm.at[pl.ds(base - half, _GPW)], idx_v)

        pltpu.async_copy(table_hbm.at[idx_v], rows_v, sem).wait()
        pltpu.sync_copy(rows_v, out_hbm.at[pl.ds(base, _GPW)])

    return _sc_gather


_HALF = N_NODES // _SC_CORES
_ACC_ROWS = _HALF + 8
_EPT = N_EDGES // _SC_SUBCORES
_ZPT = _ACC_ROWS // _SC_SUBCORES
_CPT = _HALF // _SC_SUBCORES


_ZCH = 64


@functools.cache
def _sc_scatter_kernel():
    mesh = plsc.VectorSubcoreMesh(core_axis_name="c", subcore_axis_name="s")

    @functools.partial(
        pl.kernel,
        mesh=mesh,
        out_type=jax.ShapeDtypeStruct((N_NODES, DIM), jnp.float32),
        scratch_types=[
            pltpu.VMEM((_EPT,), jnp.int32),
            pltpu.VMEM((_EPT,), jnp.int32),
            pltpu.VMEM((_EPT, DIM), jnp.float32),
            pltpu.VMEM((_ZCH, DIM), jnp.float32),
            pltpu.VMEM_SHARED((_ACC_ROWS, DIM), jnp.float32),
            pltpu.SemaphoreType.DMA,
            pltpu.SemaphoreType.DMA,
            pltpu.SemaphoreType.DMA,
        ],
    )
    def _sc_scatter(msg_hbm, tgt_hbm, out_hbm, idx_v, idx2_v, rows_v, zbuf_v,
                    acc_sh, sem_i, sem_m, sem_z):
        c = lax.axis_index("c")
        s = lax.axis_index("s")

        base = s * _EPT
        cp_i = pltpu.async_copy(tgt_hbm.at[pl.ds(base, _EPT)], idx_v, sem_i)
        cp_m = pltpu.async_copy(msg_hbm.at[pl.ds(base, _EPT)], rows_v, sem_m)

        def _zrow(i, carry):
            for j in range(DIM // 16):
                zbuf_v[i, pl.ds(j * 16, 16)] = jnp.zeros((16,), jnp.float32)
            return carry
        lax.fori_loop(0, _ZCH, _zrow, 0)
        zc = []
        for kk in range(_ZPT // _ZCH):
            zc.append(pltpu.async_copy(
                zbuf_v, acc_sh.at[pl.ds(s * _ZPT + kk * _ZCH, _ZCH)], sem_z))
        rem = _ZPT % _ZCH
        zc.append(pltpu.async_copy(
            zbuf_v.at[pl.ds(0, rem)],
            acc_sh.at[pl.ds(s * _ZPT + _ZPT - rem, rem)], sem_z))

        cp_i.wait()
        lo = c * _HALF
        for j in range(_EPT // 16):
            v = idx_v[pl.ds(j * 16, 16)] - lo
            inr = (v >= 0) & (v < _HALF)
            idx2_v[pl.ds(j * 16, 16)] = jnp.where(inr, v, _HALF)

        for z in zc:
            z.wait()
        cp_m.wait()
        plsc.subcore_barrier()
        pltpu.sync_copy(rows_v, acc_sh.at[idx2_v], add=True)
        plsc.subcore_barrier()

        pltpu.sync_copy(acc_sh.at[pl.ds(s * _CPT, _CPT)],
                        out_hbm.at[pl.ds(lo + s * _CPT, _CPT)])

        @pl.when(s == _SC_SUBCORES - 1)
        def _():
            rem = _HALF - _SC_SUBCORES * _CPT
            pltpu.sync_copy(acc_sh.at[pl.ds(_SC_SUBCORES * _CPT, rem)],
                            out_hbm.at[pl.ds(lo + _SC_SUBCORES * _CPT, rem)])

    return _sc_scatter


def _edge_mlp_body(rows_ref, rel_ref, w1a, w1b, b1, w2, b2, out_ref):
    xs = rows_ref[:N_EDGES, :].astype(jnp.bfloat16)
    xt = rows_ref[N_EDGES:, :].astype(jnp.bfloat16)
    h = jnp.dot(xs, w1a[...].astype(jnp.bfloat16),
                preferred_element_type=jnp.float32)
    h = h + jnp.dot(xt, w1b[...].astype(jnp.bfloat16),
                    preferred_element_type=jnp.float32)
    h = jnp.maximum(h + b1[...], 0.0).astype(jnp.bfloat16)
    msg = jnp.dot(h, w2[...].astype(jnp.bfloat16),
                  preferred_element_type=jnp.float32) + b2[...]
    src = rel_ref[:, 0:1]
    tgt = rel_ref[:, 2:3]
    valid = ((src < N_NODES) & (tgt < N_NODES)).astype(jnp.float32)
    out_ref[...] = msg * valid


_EB = N_EDGES // 2


def _edge_mlp_blk_body(xs_ref, xt_ref, rel_ref, w1a, w1b, b1, w2, b2, out_ref):
    xs = xs_ref[...].astype(jnp.bfloat16)
    xt = xt_ref[...].astype(jnp.bfloat16)
    h = jnp.dot(xs, w1a[...].astype(jnp.bfloat16),
                preferred_element_type=jnp.float32)
    h = h + jnp.dot(xt, w1b[...].astype(jnp.bfloat16),
                    preferred_element_type=jnp.float32)
    h = jnp.maximum(h + b1[...], 0.0).astype(jnp.bfloat16)
    msg = jnp.dot(h, w2[...].astype(jnp.bfloat16),
                  preferred_element_type=jnp.float32) + b2[...]
    src = rel_ref[:, 0:1]
    tgt = rel_ref[:, 2:3]
    valid = ((src < N_NODES) & (tgt < N_NODES)).astype(jnp.float32)
    out_ref[...] = msg * valid


def _edge_mlp(rows, relations, W1, b1, W2, b2):
    full = lambda j: (0, 0)
    nb = N_EDGES // _EB
    return pl.pallas_call(
        _edge_mlp_blk_body,
        grid=(nb,),
        in_specs=[
            pl.BlockSpec((_EB, DIM), lambda j: (j, 0)),
            pl.BlockSpec((_EB, DIM), lambda j: (j + nb, 0)),
            pl.BlockSpec((_EB, 3), lambda j: (j, 0)),
            pl.BlockSpec((DIM, DIM), full),
            pl.BlockSpec((DIM, DIM), full),
            pl.BlockSpec((1, DIM), full),
            pl.BlockSpec((DIM, DIM), full),
            pl.BlockSpec((1, DIM), full),
        ],
        out_specs=pl.BlockSpec((_EB, DIM), lambda j: (j, 0)),
        out_shape=jax.ShapeDtypeStruct((N_EDGES, DIM), jnp.float32),
    )(rows, rows, relations, W1[:DIM], W1[DIM:], b1.reshape(1, DIM), W2,
      b2.reshape(1, DIM))


def _gru_body(agg_ref, x_ref, wih, bih, whh, bhh, out_ref):
    gi = jnp.dot(agg_ref[...].astype(jnp.bfloat16),
                 wih[...].astype(jnp.bfloat16),
                 preferred_element_type=jnp.float32) + bih[...]
    gh = jnp.dot(x_ref[...].astype(jnp.bfloat16),
                 whh[...].astype(jnp.bfloat16),
                 preferred_element_type=jnp.float32) + bhh[...]
    r = jax.nn.sigmoid(gi[:, :DIM] + gh[:, :DIM])
    z = jax.nn.sigmoid(gi[:, DIM:2 * DIM] + gh[:, DIM:2 * DIM])
    n = jnp.tanh(gi[:, 2 * DIM:] + r * gh[:, 2 * DIM:])
    x32 = x_ref[...].astype(jnp.float32)
    out_ref[...] = ((1.0 - z) * n + z * x32).astype(out_ref.dtype)


def _gru(agg, x, Wih, bih, Whh, bhh, out_dtype=jnp.float32):
    R = 2000
    full = lambda i: (0, 0)
    return pl.pallas_call(
        _gru_body,
        grid=(N_NODES // R,),
        in_specs=[
            pl.BlockSpec((R, DIM), lambda i: (i, 0)),
            pl.BlockSpec((R, DIM), lambda i: (i, 0)),
            pl.BlockSpec((DIM, 3 * DIM), full),
            pl.BlockSpec((1, 3 * DIM), full),
            pl.BlockSpec((DIM, 3 * DIM), full),
            pl.BlockSpec((1, 3 * DIM), full),
        ],
        out_specs=pl.BlockSpec((R, DIM), lambda i: (i, 0)),
        out_shape=jax.ShapeDtypeStruct((N_NODES, DIM), out_dtype),
    )(agg, x, Wih, bih.reshape(1, -1), Whh, bhh.reshape(1, -1))


def _layer_norm(h, g, b):
    mu = jnp.mean(h, axis=-1, keepdims=True)
    var = jnp.mean((h - mu) ** 2, axis=-1, keepdims=True)
    return (h - mu) * jax.lax.rsqrt(var + 1e-5) * g + b


def _readout_body(sym_ref, x_ref, d1, db1, g1, c1, d2, db2, g2, c2, d3, db3,
                  out_ref):
    agg = jnp.dot(sym_ref[...].astype(jnp.bfloat16),
                  x_ref[...].astype(jnp.bfloat16),
                  preferred_element_type=jnp.float32)
    h = jnp.dot(agg.astype(jnp.bfloat16), d1[...].astype(jnp.bfloat16),
                preferred_element_type=jnp.float32) + db1[...]
    h = jnp.maximum(_layer_norm(h, g1[...], c1[...]), 0.0)
    h = jnp.dot(h.astype(jnp.bfloat16), d2[...].astype(jnp.bfloat16),
                preferred_element_type=jnp.float32) + db2[...]
    h = jnp.maximum(_layer_norm(h, g2[...], c2[...]), 0.0)
    out_ref[...] = jnp.dot(h.astype(jnp.bfloat16),
                           d3[...].astype(jnp.bfloat16),
                           preferred_element_type=jnp.float32) + db3[...]


def _readout(symbols, x, p):
    B = symbols.shape[0]
    return pl.pallas_call(
        _readout_body,
        out_shape=jax.ShapeDtypeStruct((B, DIM), jnp.float32),
    )(symbols, x,
      p["D1"], p["db1"].reshape(1, -1), p["ln1_g"].reshape(1, -1),
      p["ln1_b"].reshape(1, -1),
      p["D2"], p["db2"].reshape(1, -1), p["ln2_g"].reshape(1, -1),
      p["ln2_b"].reshape(1, -1),
      p["D3"], p["db3"].reshape(1, -1))


def kernel(symbols, relations, params):
    p = params
    x = p["emb"]
    src = relations[:, 0]
    tgt = relations[:, 2]
    for i in range(3):
        rows = _sc_gather_kernel(jnp.dtype(x.dtype).name)(x, src, tgt)
        msg = _edge_mlp(rows, relations, p[f"g{i}_W1"], p[f"g{i}_b1"],
                        p[f"g{i}_W2"], p[f"g{i}_b2"])
        agg = _sc_scatter_kernel()(msg, tgt)
        odt = jnp.bfloat16 if i == 2 else jnp.float32
        x = _gru(agg, x, p[f"g{i}_Wih"], p[f"g{i}_bih"], p[f"g{i}_Whh"],
                 p[f"g{i}_bhh"], out_dtype=odt)
    return _readout(symbols, x, p)


# speedup vs baseline: 3.1702x; 1.0012x over previous
import functools

import jax
import jax.numpy as jnp
from jax import lax
from jax.experimental import pallas as pl
from jax.experimental.pallas import tpu as pltpu
from jax.experimental.pallas import tpu_sc as plsc

N_NODES = 10000
N_EDGES = 2048
DIM = 128

_SC_CORES = 2
_SC_SUBCORES = 16
_NW = _SC_CORES * _SC_SUBCORES

_GB = 2 * N_EDGES
_GPW = _GB // _NW


@functools.cache
def _sc_gather_kernel(dt):
    dt = jnp.dtype(dt)
    mesh = plsc.VectorSubcoreMesh(core_axis_name="c", subcore_axis_name="s")

    @functools.partial(
        pl.kernel,
        mesh=mesh,
        out_type=jax.ShapeDtypeStruct((_GB, DIM), dt),
        scratch_types=[
            pltpu.VMEM((_GPW,), jnp.int32),
            pltpu.VMEM((_GPW, DIM), dt),
            pltpu.SemaphoreType.DMA,
        ],
    )
    def _sc_gather(table_hbm, src_hbm, tgt_hbm, out_hbm, idx_v, rows_v, sem):
        wid = lax.axis_index("s") * _SC_CORES + lax.axis_index("c")
        base = wid * _GPW
        half = _GB // 2

        @pl.when(base < half)
        def _():
            pltpu.sync_copy(src_hbm.at[pl.ds(base, _GPW)], idx_v)

        @pl.when(base >= half)
        def _():
            pltpu.sync_copy(tgt_hbm.at[pl.ds(base - half, _GPW)], idx_v)

        pltpu.async_copy(table_hbm.at[idx_v], rows_v, sem).wait()
        pltpu.sync_copy(rows_v, out_hbm.at[pl.ds(base, _GPW)])

    return _sc_gather


_HALF = N_NODES // _SC_CORES
_ACC_ROWS = _HALF + 8
_EPT = N_EDGES // _SC_SUBCORES
_ZPT = _ACC_ROWS // _SC_SUBCORES
_CPT = _HALF // _SC_SUBCORES


_ZCH = 64


@functools.cache
def _sc_scatter_kernel():
    mesh = plsc.VectorSubcoreMesh(core_axis_name="c", subcore_axis_name="s")

    @functools.partial(
        pl.kernel,
        mesh=mesh,
        out_type=jax.ShapeDtypeStruct((N_NODES, DIM), jnp.float32),
        scratch_types=[
            pltpu.VMEM((_EPT,), jnp.int32),
            pltpu.VMEM((_EPT,), jnp.int32),
            pltpu.VMEM((_EPT, DIM), jnp.float32),
            pltpu.VMEM((_ZCH, DIM), jnp.float32),
            pltpu.VMEM_SHARED((_ACC_ROWS, DIM), jnp.float32),
            pltpu.SemaphoreType.DMA,
            pltpu.SemaphoreType.DMA,
            pltpu.SemaphoreType.DMA,
        ],
    )
    def _sc_scatter(msg_hbm, tgt_hbm, out_hbm, idx_v, idx2_v, rows_v, zbuf_v,
                    acc_sh, sem_i, sem_m, sem_z):
        c = lax.axis_index("c")
        s = lax.axis_index("s")

        base = s * _EPT
        cp_i = pltpu.async_copy(tgt_hbm.at[pl.ds(base, _EPT)], idx_v, sem_i)
        cp_m = pltpu.async_copy(msg_hbm.at[pl.ds(base, _EPT)], rows_v, sem_m)

        def _zrow(i, carry):
            for j in range(DIM // 16):
                zbuf_v[i, pl.ds(j * 16, 16)] = jnp.zeros((16,), jnp.float32)
            return carry
        lax.fori_loop(0, _ZCH, _zrow, 0)
        zc = []
        for kk in range(_ZPT // _ZCH):
            zc.append(pltpu.async_copy(
                zbuf_v, acc_sh.at[pl.ds(s * _ZPT + kk * _ZCH, _ZCH)], sem_z))
        rem = _ZPT % _ZCH
        zc.append(pltpu.async_copy(
            zbuf_v.at[pl.ds(0, rem)],
            acc_sh.at[pl.ds(s * _ZPT + _ZPT - rem, rem)], sem_z))

        cp_i.wait()
        lo = c * _HALF
        for j in range(_EPT // 16):
            v = idx_v[pl.ds(j * 16, 16)] - lo
            inr = (v >= 0) & (v < _HALF)
            idx2_v[pl.ds(j * 16, 16)] = jnp.where(inr, v, _HALF)

        for z in zc:
            z.wait()
        cp_m.wait()
        plsc.subcore_barrier()
        pltpu.sync_copy(rows_v, acc_sh.at[idx2_v], add=True)
        plsc.subcore_barrier()

        pltpu.sync_copy(acc_sh.at[pl.ds(s * _CPT, _CPT)],
                        out_hbm.at[pl.ds(lo + s * _CPT, _CPT)])

        @pl.when(s == _SC_SUBCORES - 1)
        def _():
            rem = _HALF - _SC_SUBCORES * _CPT
            pltpu.sync_copy(acc_sh.at[pl.ds(_SC_SUBCORES * _CPT, rem)],
                            out_hbm.at[pl.ds(lo + _SC_SUBCORES * _CPT, rem)])

    return _sc_scatter


_EB = N_EDGES // 2


def _edge_mlp_blk_body(xs_ref, xt_ref, rel_ref, w1a, w1b, b1, w2, b2, out_ref):
    xs = xs_ref[...].astype(jnp.bfloat16)
    xt = xt_ref[...].astype(jnp.bfloat16)
    h = jnp.dot(xs, w1a[...].astype(jnp.bfloat16),
                preferred_element_type=jnp.float32)
    h = h + jnp.dot(xt, w1b[...].astype(jnp.bfloat16),
                    preferred_element_type=jnp.float32)
    h = jnp.maximum(h + b1[...], 0.0).astype(jnp.bfloat16)
    msg = jnp.dot(h, w2[...].astype(jnp.bfloat16),
                  preferred_element_type=jnp.float32) + b2[...]
    src = rel_ref[:, 0:1]
    tgt = rel_ref[:, 2:3]
    valid = ((src < N_NODES) & (tgt < N_NODES)).astype(jnp.float32)
    out_ref[...] = msg * valid


def _edge_mlp(rows, relations, W1, b1, W2, b2):
    full = lambda j: (0, 0)
    nb = N_EDGES // _EB
    return pl.pallas_call(
        _edge_mlp_blk_body,
        grid=(nb,),
        in_specs=[
            pl.BlockSpec((_EB, DIM), lambda j: (j, 0)),
            pl.BlockSpec((_EB, DIM), lambda j: (j + nb, 0)),
            pl.BlockSpec((_EB, 3), lambda j: (j, 0)),
            pl.BlockSpec((DIM, DIM), full),
            pl.BlockSpec((DIM, DIM), full),
            pl.BlockSpec((1, DIM), full),
            pl.BlockSpec((DIM, DIM), full),
            pl.BlockSpec((1, DIM), full),
        ],
        out_specs=pl.BlockSpec((_EB, DIM), lambda j: (j, 0)),
        out_shape=jax.ShapeDtypeStruct((N_EDGES, DIM), jnp.float32),
    )(rows, rows, relations, W1[:DIM], W1[DIM:], b1.reshape(1, DIM), W2,
      b2.reshape(1, DIM))


def _gru_body(agg_ref, x_ref, wih, bih, whh, bhh, out_ref):
    gi = jnp.dot(agg_ref[...].astype(jnp.bfloat16),
                 wih[...].astype(jnp.bfloat16),
                 preferred_element_type=jnp.float32) + bih[...]
    gh = jnp.dot(x_ref[...].astype(jnp.bfloat16),
                 whh[...].astype(jnp.bfloat16),
                 preferred_element_type=jnp.float32) + bhh[...]
    r = jax.nn.sigmoid(gi[:, :DIM] + gh[:, :DIM])
    z = jax.nn.sigmoid(gi[:, DIM:2 * DIM] + gh[:, DIM:2 * DIM])
    n = jnp.tanh(gi[:, 2 * DIM:] + r * gh[:, 2 * DIM:])
    x32 = x_ref[...].astype(jnp.float32)
    out_ref[...] = ((1.0 - z) * n + z * x32).astype(out_ref.dtype)


def _gru(agg, x, Wih, bih, Whh, bhh, out_dtype=jnp.float32):
    R = 2000
    full = lambda i: (0, 0)
    return pl.pallas_call(
        _gru_body,
        grid=(N_NODES // R,),
        in_specs=[
            pl.BlockSpec((R, DIM), lambda i: (i, 0)),
            pl.BlockSpec((R, DIM), lambda i: (i, 0)),
            pl.BlockSpec((DIM, 3 * DIM), full),
            pl.BlockSpec((1, 3 * DIM), full),
            pl.BlockSpec((DIM, 3 * DIM), full),
            pl.BlockSpec((1, 3 * DIM), full),
        ],
        out_specs=pl.BlockSpec((R, DIM), lambda i: (i, 0)),
        out_shape=jax.ShapeDtypeStruct((N_NODES, DIM), out_dtype),
    )(agg, x, Wih, bih.reshape(1, -1), Whh, bhh.reshape(1, -1))


def _layer_norm(h, g, b):
    mu = jnp.mean(h, axis=-1, keepdims=True)
    var = jnp.mean((h - mu) ** 2, axis=-1, keepdims=True)
    return (h - mu) * jax.lax.rsqrt(var + 1e-5) * g + b


def _readout_body(sym_ref, x_ref, d1, db1, g1, c1, d2, db2, g2, c2, d3, db3,
                  out_ref):
    agg = jnp.dot(sym_ref[...].astype(jnp.bfloat16),
                  x_ref[...].astype(jnp.bfloat16),
                  preferred_element_type=jnp.float32)
    h = jnp.dot(agg.astype(jnp.bfloat16), d1[...].astype(jnp.bfloat16),
                preferred_element_type=jnp.float32) + db1[...]
    h = jnp.maximum(_layer_norm(h, g1[...], c1[...]), 0.0)
    h = jnp.dot(h.astype(jnp.bfloat16), d2[...].astype(jnp.bfloat16),
                preferred_element_type=jnp.float32) + db2[...]
    h = jnp.maximum(_layer_norm(h, g2[...], c2[...]), 0.0)
    out_ref[...] = jnp.dot(h.astype(jnp.bfloat16),
                           d3[...].astype(jnp.bfloat16),
                           preferred_element_type=jnp.float32) + db3[...]


def _readout(symbols, x, p):
    B = symbols.shape[0]
    return pl.pallas_call(
        _readout_body,
        out_shape=jax.ShapeDtypeStruct((B, DIM), jnp.float32),
    )(symbols, x,
      p["D1"], p["db1"].reshape(1, -1), p["ln1_g"].reshape(1, -1),
      p["ln1_b"].reshape(1, -1),
      p["D2"], p["db2"].reshape(1, -1), p["ln2_g"].reshape(1, -1),
      p["ln2_b"].reshape(1, -1),
      p["D3"], p["db3"].reshape(1, -1))


def kernel(symbols, relations, params):
    p = params
    x = p["emb"]
    src = relations[:, 0]
    tgt = relations[:, 2]
    for i in range(3):
        rows = _sc_gather_kernel(jnp.dtype(x.dtype).name)(x, src, tgt)
        msg = _edge_mlp(rows, relations, p[f"g{i}_W1"], p[f"g{i}_b1"],
                        p[f"g{i}_W2"], p[f"g{i}_b2"])
        agg = _sc_scatter_kernel()(msg, tgt)
        odt = jnp.bfloat16 if i == 2 else jnp.float32
        x = _gru(agg, x, p[f"g{i}_Wih"], p[f"g{i}_bih"], p[f"g{i}_Whh"],
                 p[f"g{i}_bhh"], out_dtype=odt)
    return _readout(symbols, x, p)
